# baseline (speedup 1.0000x reference)
import jax
import jax.numpy as jnp
from jax.experimental import pallas as pl
from jax.experimental.pallas import tpu as pltpu

EPS_DICE = 0.0001


def _dice_block_kernel(out_ref, tgt_ref, loss_ref):
    c, h, w = out_ref.shape[1:]
    x = out_ref[0]
    t = tgt_ref[0]
    hs = 32
    cls = jax.lax.broadcasted_iota(jnp.int32, (c, hs, w), 0)
    tot_parts = []
    s2_parts = []
    cnt_parts = []
    for k in range(h // hs):
        xk = x[:, k * hs : (k + 1) * hs, :]
        tk = t[None, k * hs : (k + 1) * hs, :]
        th = jnp.tanh(xk * 0.5)
        m = tk == cls
        tot_parts.append(jnp.sum(th, axis=(1, 2)))
        s2_parts.append(jnp.sum(jnp.where(m, th + 1.0, 0.0), axis=(1, 2)))
        cnt_parts.append(jnp.sum(jnp.where(m, 1.0, 0.0), axis=(1, 2)))
    t_tot = sum(tot_parts)
    s2 = sum(s2_parts)
    cnt = sum(cnt_parts)
    sig_sum = 0.5 * t_tot + 0.5 * jnp.float32(h * w)
    numer = s2
    loss_pc = 1.0 - numer / (sig_sum + cnt + EPS_DICE)
    loss_ref[0] = (jnp.sum(loss_pc, keepdims=True) * (1.0 / c)).reshape(1, 1)


@jax.jit
def kernel(output, target):
    b, c, h, w = output.shape
    tgt = target.astype(jnp.int32)
    loss = pl.pallas_call(
        _dice_block_kernel,
        grid=(b,),
        in_specs=[
            pl.BlockSpec((1, c, h, w), lambda i: (i, 0, 0, 0)),
            pl.BlockSpec((1, h, w), lambda i: (i, 0, 0)),
        ],
        out_specs=pl.BlockSpec((1, 1, 1), lambda i: (i, 0, 0)),
        out_shape=jax.ShapeDtypeStruct((b, 1, 1), jnp.float32),
        compiler_params=pltpu.CompilerParams(
            dimension_semantics=("arbitrary",),
            vmem_limit_bytes=100 * 1024 * 1024,
        ),
    )(output, tgt)
    return loss[:, 0, 0]
